# Initial kernel scaffold; baseline (speedup 1.0000x reference)
import jax
import jax.numpy as jnp
from jax.experimental import pallas as pl


def kernel(x, W, b, centers):
    raise NotImplementedError("write your pallas kernel here")


# speedup vs baseline: 25.0176x; 25.0176x over previous
import functools

import jax
import jax.numpy as jnp
from jax import lax
from jax.experimental import pallas as pl
from jax.experimental.pallas import tpu as pltpu

DIM = 64
CB = 256
NCB = 16
NLOG = CB * NCB
TOK = 16384
BT = 512


def _tc_body(x_ref, w_ref, b_ref, c_ref, idx_ref, rec_ref):
    xt = x_ref[...]
    w = w_ref[...]
    logits = lax.dot_general(
        xt, w, (((1,), (1,)), ((), ())), preferred_element_type=jnp.float32)
    logits = logits + b_ref[...]
    rec = jnp.zeros((BT, DIM), jnp.float32)
    cols = []
    lane = lax.broadcasted_iota(jnp.int32, (BT, CB), 1)
    for j in range(NCB):
        blk = logits[:, j * CB:(j + 1) * CB]
        idxj = jnp.argmax(blk, axis=1).astype(jnp.int32)
        cols.append(idxj[:, None])
        oh = (lane == idxj[:, None]).astype(jnp.float32)
        rec = rec + lax.dot_general(
            oh, c_ref[j * CB:(j + 1) * CB, :], (((1,), (0,)), ((), ())),
            preferred_element_type=jnp.float32)
    idx_ref[...] = jnp.concatenate(cols, axis=1)
    rec_ref[...] = rec


@functools.partial(jax.jit, static_argnames=("interpret",))
def _encode(x, W, b, centers, interpret=False):
    b2 = b.reshape(1, NLOG)
    grid = (TOK // BT,)
    return pl.pallas_call(
        _tc_body,
        grid=grid,
        in_specs=[
            pl.BlockSpec((BT, DIM), lambda i: (i, 0)),
            pl.BlockSpec((NLOG, DIM), lambda i: (0, 0)),
            pl.BlockSpec((1, NLOG), lambda i: (0, 0)),
            pl.BlockSpec((NLOG, DIM), lambda i: (0, 0)),
        ],
        out_specs=[
            pl.BlockSpec((BT, NCB), lambda i: (i, 0)),
            pl.BlockSpec((BT, DIM), lambda i: (i, 0)),
        ],
        out_shape=[
            jax.ShapeDtypeStruct((TOK, NCB), jnp.int32),
            jax.ShapeDtypeStruct((TOK, DIM), jnp.float32),
        ],
        interpret=interpret,
    )(x, W, b2, centers)


def kernel(x, W, b, centers):
    indexes, recon = _encode(x, W, b, centers)
    return indexes, recon
